# Initial kernel scaffold; baseline (speedup 1.0000x reference)
import jax
import jax.numpy as jnp
from jax.experimental import pallas as pl


def kernel(x):
    raise NotImplementedError("write your pallas kernel here")


# speedup vs baseline: 3.9962x; 3.9962x over previous
import jax
import jax.numpy as jnp
from jax import lax
from jax.experimental import pallas as pl
from jax.experimental.pallas import tpu as pltpu
from jax.experimental.pallas import tpu_sc as plsc

BSZ = 1024
D = 32768
K = 32
L = 16
NB = 1024
SH = 22
NV = D // L
CH = 8
CAPW = 12544
CCAP = 1024
NC = 2
NS = 16
NW = NC * NS
ROWS_PER_W = BSZ // NW
MASK31 = 0x7FFFFFFF

assert NB * L // L * 2 == NV


def _scal(v):
    return v if v.ndim == 0 else jnp.max(v)


def _key_of(v):
    ki = plsc.bitcast(v, jnp.int32)
    return ki ^ ((ki >> 31) & MASK31)


def _inv_key_f(kv):
    return plsc.bitcast(kv ^ ((kv >> 31) & MASK31), jnp.float32)


def _sc_body(x_hbm, o_hbm, buf0, buf1, hist, ckey, cpos, ckey2, cpos2, offr,
             isem0, isem1, osem0, osem1):
    wid = lax.axis_index("s") * NC + lax.axis_index("c")
    row0 = wid * ROWS_PER_W
    lanes = lax.iota(jnp.int32, L)
    lanevec = lanes * NB + NB // 2
    ones_i = jnp.ones((L,), jnp.int32)
    zeros_i = jnp.zeros((L,), jnp.int32)
    zeros_f = jnp.zeros((L,), jnp.float32)
    intmin = jnp.full((L,), jnp.int32(-(2**31)))

    def zh(i, _):
        hist[pl.ds(i * L, L)] = zeros_i
        return 0
    lax.fori_loop(0, NB * L // L, zh, 0)

    pltpu.async_copy(x_hbm.at[row0], buf0, isem0)

    def do_row(r, buf, isem, osem, o_isem, o_osem, obuf, first, last):
        row = row0 + r
        pltpu.make_async_copy(x_hbm.at[row], buf, isem).wait()

        @plsc.parallel_loop(0, NV, 1, unroll=8, carry=jnp.int32(0))
        def p1(i, c_):
            v = buf[pl.ds(i * L, L)]
            b = _key_of(v) >> SH
            plsc.addupdate_scatter(hist, [b + lanevec], ones_i)
            return c_
        del p1

        def scond(st):
            return st[2] == 0

        def sbody(st):
            b, cum, _f, _bb, _ca = st
            h = zeros_i
            for l in range(L):
                h = h + hist[pl.ds(l * NB + b, L)]
            hr = lax.rev(h, (0,))
            cs = plsc.cumsum(hr)
            tot = cum + cs
            m = tot >= K
            anyf = _scal(plsc.all_reduce_population_count(m)) > 0
            jstar = _scal(plsc.all_reduce_ffs(m))
            at_j = lambda vec: _scal(jnp.where(lanes == jstar, vec, 0))
            found = jnp.where(anyf, jnp.int32(1), jnp.int32(0))
            bb = jnp.where(anyf, b + (L - 1) - jstar, 0)
            ca = jnp.where(anyf, at_j(tot) - at_j(hr), 0)
            return (b - L, cum + _scal(cs), found, bb, ca)

        st = lax.while_loop(
            scond, sbody,
            (jnp.int32(NB - L), jnp.int32(0), jnp.int32(0), jnp.int32(0),
             jnp.int32(0)))
        bb, cnt_above = st[3], st[4]

        if not first:
            pltpu.make_async_copy(obuf, o_hbm.at[row], o_osem).wait()
        if not last:
            pltpu.async_copy(x_hbm.at[row + 1], obuf, o_isem)

        blo_s = (bb - NB // 2) << SH
        lo_s = (bb + 1 - NB // 2) << SH
        blo_k = jnp.broadcast_to(blo_s, (L,))
        lo_k = jnp.broadcast_to(lo_s, (L,))
        blo_f = _inv_key_f(blo_k)
        lo_f = _inv_key_f(lo_k)

        offr[0] = 0

        def p2(c, _):
            vs, mbs = [], []
            acc = jnp.zeros((L,), jnp.bool_)
            for u in range(CH):
                i = c * CH + u
                v = buf[pl.ds(i * L, L)]
                keep = v >= lo_f
                buf[pl.ds(i * L, L)] = jnp.where(keep, v, zeros_f)
                if u % 2 == 0:
                    hist[pl.ds((c * (CH // 2) + u // 2) * L, L)] = zeros_i
                mb = (v >= blo_f) & jnp.logical_not(keep)
                vs.append(v)
                mbs.append(mb)
                acc = acc | mb

            @pl.when(jnp.any(acc))
            def _collect():
                for u in range(CH):
                    i = c * CH + u

                    @pl.when(jnp.any(mbs[u]))
                    def _one():
                        o = jnp.minimum(offr[0], CAPW - L)
                        ckey[pl.ds(o, L)] = _key_of(vs[u])
                        cpos[pl.ds(o, L)] = i * L + lanes
                        offr[0] = o + L
            return 0
        lax.fori_loop(0, NV // CH, p2, 0)

        nvb = offr[0] >> 4
        k2 = K - cnt_above

        def comp(v, off2):
            kv = ckey[pl.ds(v * L, L)]
            pv = cpos[pl.ds(v * L, L)]
            valid = (kv >= blo_k) & (kv < lo_k)
            vi = jnp.where(valid, 1, 0)
            pc = plsc.cumsum(vi)
            dst = jnp.minimum(off2, CCAP - L) + pc - vi
            plsc.store_scatter(ckey2, [dst], kv, mask=valid)
            plsc.store_scatter(cpos2, [dst], pv, mask=valid)
            return off2 + _scal(plsc.all_reduce_population_count(valid))
        n2 = lax.fori_loop(0, nvb, comp, jnp.int32(0))
        n2 = jnp.minimum(n2, CCAP - L)
        plsc.store_scatter(ckey2, [n2 + lanes], intmin)
        nvb2 = (n2 + (L - 1)) >> 4

        def bstep(j, prefix):
            t = prefix | (jnp.int32(1) << (SH - 1 - j))

            def cnt1(v, cacc):
                ge = ckey2[pl.ds(v * L, L)] >= t
                return cacc + _scal(plsc.all_reduce_population_count(ge))
            cnum = lax.fori_loop(0, nvb2, cnt1, jnp.int32(0))
            return jnp.where(cnum >= k2, t, prefix)
        thr = lax.fori_loop(0, SH, bstep, blo_s)

        def fix(v, _):
            kv = ckey2[pl.ds(v * L, L)]
            pv = cpos2[pl.ds(v * L, L)]
            sel = kv >= thr
            plsc.store_scatter(buf, [pv], _inv_key_f(kv), mask=sel)
            return 0
        lax.fori_loop(0, nvb2, fix, 0)

        pltpu.async_copy(buf, o_hbm.at[row], osem)

    def pair(g, _):
        do_row(2 * g, buf0, isem0, osem0, isem1, osem1, buf1,
               first=False, last=False)
        do_row(2 * g + 1, buf1, isem1, osem1, isem0, osem0, buf0,
               first=False, last=False)
        return 0

    do_row(0, buf0, isem0, osem0, isem1, osem1, buf1, first=True, last=False)
    do_row(1, buf1, isem1, osem1, isem0, osem0, buf0, first=False, last=False)
    lax.fori_loop(1, ROWS_PER_W // 2 - 1, pair, 0)
    do_row(ROWS_PER_W - 2, buf0, isem0, osem0, isem1, osem1, buf1,
           first=False, last=False)
    do_row(ROWS_PER_W - 1, buf1, isem1, osem1, isem0, osem0, buf0,
           first=False, last=True)

    pltpu.make_async_copy(buf1, o_hbm.at[row0 + ROWS_PER_W - 1], osem1).wait()


@jax.jit
def kernel(x):
    run = pl.kernel(
        _sc_body,
        out_type=jax.ShapeDtypeStruct((BSZ, D), jnp.float32),
        mesh=plsc.VectorSubcoreMesh(core_axis_name="c", subcore_axis_name="s"),
        compiler_params=pltpu.CompilerParams(needs_layout_passes=False),
        scratch_types=[
            pltpu.VMEM((D,), jnp.float32),
            pltpu.VMEM((D,), jnp.float32),
            pltpu.VMEM((NB * L,), jnp.int32),
            pltpu.VMEM((CAPW,), jnp.int32),
            pltpu.VMEM((CAPW,), jnp.int32),
            pltpu.VMEM((CCAP + L,), jnp.int32),
            pltpu.VMEM((CCAP + L,), jnp.int32),
            pltpu.SMEM((1,), jnp.int32),
            pltpu.SemaphoreType.DMA,
            pltpu.SemaphoreType.DMA,
            pltpu.SemaphoreType.DMA,
            pltpu.SemaphoreType.DMA,
        ],
    )
    return run(x)
